# Initial kernel scaffold; baseline (speedup 1.0000x reference)
import jax
import jax.numpy as jnp
from jax.experimental import pallas as pl


def kernel(images, row_table, col_table):
    raise NotImplementedError("write your pallas kernel here")


# speedup vs baseline: 1.2133x; 1.2133x over previous
import functools

import jax
import jax.numpy as jnp
import numpy as np
from jax import lax
from jax.experimental import pallas as pl
from jax.experimental.pallas import tpu as pltpu
from jax.experimental.pallas import tpu_sc as plsc

_VOCAB_SIZE = 128
_PATCH_SIZE = 16
_LANES = 16


def _rotl(x, r):
    return ((x << np.uint32(r)) | (x >> np.uint32(32 - r))).astype(np.uint32)


def _threefry2x32(k0, k1, x0, x1):
    rotations = [(13, 15, 26, 6), (17, 29, 16, 24)]
    ks = [np.uint32(k0), np.uint32(k1),
          np.uint32(np.uint32(k0) ^ np.uint32(k1) ^ np.uint32(0x1BD11BDA))]
    x = [(x0 + ks[0]).astype(np.uint32), (x1 + ks[1]).astype(np.uint32)]
    for i in range(5):
        for r in rotations[i % 2]:
            x[0] = (x[0] + x[1]).astype(np.uint32)
            x[1] = _rotl(x[1], r)
            x[1] = x[1] ^ x[0]
        x[0] = (x[0] + ks[(i + 1) % 3]).astype(np.uint32)
        x[1] = (x[1] + ks[(i + 2) % 3] + np.uint32(i + 1)).astype(np.uint32)
    return x


def _np_split2(k0, k1):
    b1, b2 = _threefry2x32(k0, k1, np.zeros(2, np.uint32),
                           np.arange(2, dtype=np.uint32))
    return [(b1[0], b2[0]), (b1[1], b2[1])]


def _np_uniform(k0, k1, shape):
    n = int(np.prod(shape))
    b1, b2 = _threefry2x32(k0, k1, np.zeros(n, np.uint32),
                           np.arange(n, dtype=np.uint32))
    bits = b1 ^ b2
    floats = ((bits >> np.uint32(9)) | np.uint32(0x3F800000)).view(np.float32)
    floats = floats - np.float32(1.0)
    return np.maximum(np.float32(0), floats).reshape(shape)


@functools.lru_cache(maxsize=None)
def _sampled_indices(batch_size: int, n_rows: int, n_cols: int):
    qr = np.round(np.arange(n_rows + 1, dtype=np.float32) / np.float32(n_rows)
                  * np.float32(_VOCAB_SIZE)).astype(np.int32)
    qc = np.round(np.arange(n_cols + 1, dtype=np.float32) / np.float32(n_cols)
                  * np.float32(_VOCAB_SIZE)).astype(np.int32)
    rw = (qr[1:] - qr[:-1]).astype(np.float32)
    cw = (qc[1:] - qc[:-1]).astype(np.float32)
    (kr0, kr1), (kc0, kc1) = _np_split2(np.uint32(0), np.uint32(42))
    ur = _np_uniform(kr0, kr1, (batch_size, n_rows))
    uc = _np_uniform(kc0, kc1, (batch_size, n_cols))
    ridx = qr[:-1][None, :] + np.floor(ur * rw[None, :]).astype(np.int32)
    cidx = qc[:-1][None, :] + np.floor(uc * cw[None, :]).astype(np.int32)
    return ridx, cidx


@functools.lru_cache(maxsize=None)
def _make_sc_kernel(batch_size: int, n_rows: int, n_cols: int):
    info = plsc.get_sparse_core_info()
    nc, ns = info.num_cores, info.num_subcores
    nw = nc * ns
    assert batch_size % nw == 0
    mesh = plsc.VectorSubcoreMesh(core_axis_name="c", subcore_axis_name="s")

    @functools.partial(
        pl.kernel,
        mesh=mesh,
        compiler_params=pltpu.CompilerParams(needs_layout_passes=False),
        out_type=jax.ShapeDtypeStruct((batch_size, n_rows, n_cols), jnp.float32),
        scratch_types=[
            pltpu.VMEM((_VOCAB_SIZE,), jnp.float32),
            pltpu.VMEM((_VOCAB_SIZE,), jnp.float32),
            pltpu.VMEM((n_rows,), jnp.int32),
            pltpu.VMEM((n_cols,), jnp.int32),
            pltpu.VMEM((n_rows + _LANES,), jnp.float32),
            pltpu.VMEM((n_rows, n_cols), jnp.float32),
        ],
    )
    def sc_kernel(rt_hbm, ct_hbm, ridx_hbm, cidx_hbm, out_hbm,
                  rt_v, ct_v, ridx_v, cidx_v, rvals_v, out_v):
        wid = lax.axis_index("s") * nc + lax.axis_index("c")
        pltpu.sync_copy(rt_hbm, rt_v)
        pltpu.sync_copy(ct_hbm, ct_v)
        for b0 in range(0, batch_size, nw):
            b = b0 + wid
            pltpu.sync_copy(ridx_hbm.at[b], ridx_v)
            pltpu.sync_copy(cidx_hbm.at[b], cidx_v)
            cvals = []
            for j in range(0, n_cols, _LANES):
                cvals.append(plsc.load_gather(ct_v, [cidx_v[pl.ds(j, _LANES)]]))
            for j in range(0, n_rows, _LANES):
                rvals_v[pl.ds(_LANES + j, _LANES)] = plsc.load_gather(
                    rt_v, [ridx_v[pl.ds(j, _LANES)]]
                )
            for r in range(n_rows):
                rv = plsc.load_gather(
                    rvals_v, [jnp.full((_LANES,), _LANES + r, jnp.int32)]
                )
                for j in range(0, n_cols, _LANES):
                    out_v[r, pl.ds(j, _LANES)] = rv + cvals[j // _LANES]
            pltpu.sync_copy(out_v, out_hbm.at[b])

    return sc_kernel


def kernel(images, row_table, col_table):
    batch_size, _, height, width = images.shape
    n_rows = height // _PATCH_SIZE
    n_cols = width // _PATCH_SIZE
    ridx, cidx = _sampled_indices(batch_size, n_rows, n_cols)
    sc = _make_sc_kernel(batch_size, n_rows, n_cols)
    out = sc(
        row_table.reshape(_VOCAB_SIZE),
        col_table.reshape(_VOCAB_SIZE),
        jnp.asarray(ridx),
        jnp.asarray(cidx),
    )
    return out.reshape(batch_size, 1, n_rows, n_cols)


# speedup vs baseline: 1.2801x; 1.0550x over previous
import functools

import jax
import jax.numpy as jnp
import numpy as np
from jax import lax
from jax.experimental import pallas as pl
from jax.experimental.pallas import tpu as pltpu
from jax.experimental.pallas import tpu_sc as plsc

_VOCAB_SIZE = 128
_PATCH_SIZE = 16
_LANES = 16


def _rotl(x, r):
    return ((x << np.uint32(r)) | (x >> np.uint32(32 - r))).astype(np.uint32)


def _threefry2x32(k0, k1, x0, x1):
    rotations = [(13, 15, 26, 6), (17, 29, 16, 24)]
    ks = [np.uint32(k0), np.uint32(k1),
          np.uint32(np.uint32(k0) ^ np.uint32(k1) ^ np.uint32(0x1BD11BDA))]
    x = [(x0 + ks[0]).astype(np.uint32), (x1 + ks[1]).astype(np.uint32)]
    for i in range(5):
        for r in rotations[i % 2]:
            x[0] = (x[0] + x[1]).astype(np.uint32)
            x[1] = _rotl(x[1], r)
            x[1] = x[1] ^ x[0]
        x[0] = (x[0] + ks[(i + 1) % 3]).astype(np.uint32)
        x[1] = (x[1] + ks[(i + 2) % 3] + np.uint32(i + 1)).astype(np.uint32)
    return x


def _np_split2(k0, k1):
    b1, b2 = _threefry2x32(k0, k1, np.zeros(2, np.uint32),
                           np.arange(2, dtype=np.uint32))
    return [(b1[0], b2[0]), (b1[1], b2[1])]


def _np_uniform(k0, k1, shape):
    n = int(np.prod(shape))
    b1, b2 = _threefry2x32(k0, k1, np.zeros(n, np.uint32),
                           np.arange(n, dtype=np.uint32))
    bits = b1 ^ b2
    floats = ((bits >> np.uint32(9)) | np.uint32(0x3F800000)).view(np.float32)
    floats = floats - np.float32(1.0)
    return np.maximum(np.float32(0), floats).reshape(shape)


@functools.lru_cache(maxsize=None)
def _sampled_indices(batch_size: int, n_rows: int, n_cols: int):
    qr = np.round(np.arange(n_rows + 1, dtype=np.float32) / np.float32(n_rows)
                  * np.float32(_VOCAB_SIZE)).astype(np.int32)
    qc = np.round(np.arange(n_cols + 1, dtype=np.float32) / np.float32(n_cols)
                  * np.float32(_VOCAB_SIZE)).astype(np.int32)
    rw = (qr[1:] - qr[:-1]).astype(np.float32)
    cw = (qc[1:] - qc[:-1]).astype(np.float32)
    (kr0, kr1), (kc0, kc1) = _np_split2(np.uint32(0), np.uint32(42))
    ur = _np_uniform(kr0, kr1, (batch_size, n_rows))
    uc = _np_uniform(kc0, kc1, (batch_size, n_cols))
    ridx = qr[:-1][None, :] + np.floor(ur * rw[None, :]).astype(np.int32)
    cidx = qc[:-1][None, :] + np.floor(uc * cw[None, :]).astype(np.int32)
    return ridx, cidx


@functools.lru_cache(maxsize=None)
def _make_sc_kernel(batch_size: int, n_rows: int, n_cols: int):
    info = plsc.get_sparse_core_info()
    nc, ns = info.num_cores, info.num_subcores
    nw = nc * ns
    assert batch_size % nw == 0
    mesh = plsc.VectorSubcoreMesh(core_axis_name="c", subcore_axis_name="s")

    @functools.partial(
        pl.kernel,
        mesh=mesh,
        compiler_params=pltpu.CompilerParams(needs_layout_passes=False),
        out_type=jax.ShapeDtypeStruct((batch_size, n_rows, n_cols), jnp.float32),
        scratch_types=[
            pltpu.VMEM((_VOCAB_SIZE,), jnp.float32),
            pltpu.VMEM((_VOCAB_SIZE,), jnp.float32),
            pltpu.VMEM((n_rows + n_cols,), jnp.int32),
            pltpu.VMEM((n_rows + _LANES,), jnp.float32),
            pltpu.VMEM((n_rows, n_cols), jnp.float32),
            pltpu.SemaphoreType.DMA,
        ],
    )
    def sc_kernel(rt_hbm, ct_hbm, idx_hbm, out_hbm,
                  rt_v, ct_v, idx_v, rvals_v, out_v, sem):
        wid = lax.axis_index("s") * nc + lax.axis_index("c")
        for b0 in range(0, batch_size, nw):
            b = b0 + wid
            c1 = pltpu.async_copy(rt_hbm, rt_v, sem)
            c2 = pltpu.async_copy(ct_hbm, ct_v, sem)
            c3 = pltpu.async_copy(idx_hbm.at[b], idx_v, sem)
            c1.wait()
            c2.wait()
            c3.wait()
            cvals = []
            for j in range(0, n_cols, _LANES):
                cvals.append(
                    plsc.load_gather(ct_v, [idx_v[pl.ds(n_rows + j, _LANES)]])
                )
            for j in range(0, n_rows, _LANES):
                rvals_v[pl.ds(_LANES + j, _LANES)] = plsc.load_gather(
                    rt_v, [idx_v[pl.ds(j, _LANES)]]
                )
            for r in range(n_rows):
                rv = plsc.load_gather(
                    rvals_v, [jnp.full((_LANES,), _LANES + r, jnp.int32)]
                )
                for j in range(0, n_cols, _LANES):
                    out_v[r, pl.ds(j, _LANES)] = rv + cvals[j // _LANES]
            pltpu.sync_copy(out_v, out_hbm.at[b])

    return sc_kernel


def kernel(images, row_table, col_table):
    batch_size, _, height, width = images.shape
    n_rows = height // _PATCH_SIZE
    n_cols = width // _PATCH_SIZE
    ridx, cidx = _sampled_indices(batch_size, n_rows, n_cols)
    idx = np.concatenate([ridx, cidx], axis=1)
    sc = _make_sc_kernel(batch_size, n_rows, n_cols)
    out = sc(
        row_table.reshape(_VOCAB_SIZE),
        col_table.reshape(_VOCAB_SIZE),
        jnp.asarray(idx),
    )
    return out.reshape(batch_size, 1, n_rows, n_cols)
